# baseline (speedup 1.0000x reference)
import jax
import jax.numpy as jnp
from jax import lax
from jax.experimental import pallas as pl
from jax.experimental.pallas import tpu as pltpu
from jax.experimental.pallas import tpu_sc as plsc

_SEQ = 2048
_NOUT = 1000
_NIN = 1000
_LANES = 16
_TBLK = 512


def _sc_counters_body(seq_hbm, delta_hbm, out_hbm, seq_v, delta_v, out_v):
    cid = lax.axis_index("c")
    sid = lax.axis_index("s")

    @pl.when(jnp.logical_and(cid == 0, sid == 0))
    def _():
        pltpu.sync_copy(seq_hbm, seq_v)
        pltpu.sync_copy(delta_hbm, delta_v)

        def body(i, carry):
            idx = seq_v[pl.ds(i * _LANES, _LANES)]
            g = plsc.load_gather(delta_v, [idx])
            out_v[pl.ds(i * _LANES, _LANES)] = plsc.cumsum(g) + carry
            return carry + jnp.sum(g)

        lax.fori_loop(0, _SEQ // _LANES, body, jnp.float32(0.0))
        pltpu.sync_copy(out_v, out_hbm.at[0])


def _sc_counters(input_seq, delta):
    mesh = plsc.VectorSubcoreMesh(core_axis_name="c", subcore_axis_name="s")
    return pl.kernel(
        _sc_counters_body,
        out_type=jax.ShapeDtypeStruct((1, _SEQ), jnp.float32),
        mesh=mesh,
        scratch_types=[
            pltpu.VMEM((_SEQ,), jnp.int32),
            pltpu.VMEM((_NIN,), jnp.float32),
            pltpu.VMEM((_SEQ,), jnp.float32),
        ],
        compiler_params=pltpu.CompilerParams(
            needs_layout_passes=False, skip_device_barrier=True
        ),
    )(input_seq, delta)


def _dense_body(c_ref, w_ref, b_ref, o_ref):
    logits = w_ref[...] * c_ref[...] + b_ref[...]
    m = jnp.max(logits, axis=0, keepdims=True)
    e = jnp.exp(logits - m)
    o_ref[...] = e / jnp.sum(e, axis=0, keepdims=True)


def _dense_softmax_t(counters_row, W, bcol):
    return pl.pallas_call(
        _dense_body,
        grid=(_SEQ // _TBLK,),
        in_specs=[
            pl.BlockSpec((1, _TBLK), lambda i: (0, i)),
            pl.BlockSpec((_NOUT, 1), lambda i: (0, 0)),
            pl.BlockSpec((_NOUT, 1), lambda i: (0, 0)),
        ],
        out_specs=pl.BlockSpec((_NOUT, _TBLK), lambda i: (0, i)),
        out_shape=jax.ShapeDtypeStruct((_NOUT, _SEQ), jnp.float32),
    )(counters_row, W, bcol)


def kernel(input_seq, delta, W, b):
    counters_row = _sc_counters(input_seq, delta)
    out_t = _dense_softmax_t(counters_row, W, b[:, None])
    return out_t.T


# speedup vs baseline: 1.4042x; 1.4042x over previous
import jax
import jax.numpy as jnp
from jax import lax
from jax.experimental import pallas as pl
from jax.experimental.pallas import tpu as pltpu

_SEQ = 2048
_NOUT = 1000
_NIN = 1000
_TBLK = 512
_SUBL = 4


def _body(seq_ref, delta_ref, wb_ref, o_ref, srow, scanm, crow2, carry):
    i = pl.program_id(0)

    @pl.when(i == 0)
    def _():
        carry[0, 0] = jnp.float32(0.0)
        row = lax.broadcasted_iota(jnp.int32, (_TBLK, _TBLK), 0)
        col = lax.broadcasted_iota(jnp.int32, (_TBLK, _TBLK), 1)
        scanm[...] = (row <= col).astype(jnp.float32)
        crow2[1:2, :] = jnp.ones((1, _TBLK), jnp.float32)

    for k in range(_SUBL):
        srow[0:1, k * 128 : (k + 1) * 128] = seq_ref[0, k : k + 1, :]

    vrow = lax.broadcasted_iota(jnp.int32, (_NIN, _TBLK), 0)
    onehot = (vrow == srow[...]).astype(jnp.float32)
    g = lax.dot_general(
        delta_ref[...], onehot, (((1,), (0,)), ((), ())),
        preferred_element_type=jnp.float32,
        precision=lax.Precision.HIGHEST,
    )

    csum = lax.dot_general(
        g, scanm[...], (((1,), (0,)), ((), ())),
        preferred_element_type=jnp.float32,
        precision=lax.Precision.HIGHEST,
    )
    c0 = carry[0, 0]
    crow2[0:1, :] = csum + c0
    carry[0, 0] = c0 + jnp.sum(g)

    logits = lax.dot_general(
        wb_ref[...], crow2[...], (((1,), (0,)), ((), ())),
        preferred_element_type=jnp.float32,
        precision=lax.Precision.HIGHEST,
    )
    m = jnp.max(logits, axis=0, keepdims=True)
    e = jnp.exp(logits - m)
    o_ref[...] = e / jnp.sum(e, axis=0, keepdims=True)


def kernel(input_seq, delta, W, b):
    seq3d = input_seq.reshape(_SEQ // _TBLK, _SUBL, 128)
    wb = jnp.stack([W[:, 0], b], axis=1)
    out_t = pl.pallas_call(
        _body,
        grid=(_SEQ // _TBLK,),
        in_specs=[
            pl.BlockSpec((1, _SUBL, 128), lambda i: (i, 0, 0)),
            pl.BlockSpec((1, _NIN), lambda i: (0, 0)),
            pl.BlockSpec((_NOUT, 2), lambda i: (0, 0)),
        ],
        out_specs=pl.BlockSpec((_NOUT, _TBLK), lambda i: (0, i)),
        out_shape=jax.ShapeDtypeStruct((_NOUT, _SEQ), jnp.float32),
        scratch_shapes=[
            pltpu.VMEM((1, _TBLK), jnp.int32),
            pltpu.VMEM((_TBLK, _TBLK), jnp.float32),
            pltpu.VMEM((2, _TBLK), jnp.float32),
            pltpu.SMEM((1, 1), jnp.float32),
        ],
    )(seq3d, delta[None, :], wb)
    return out_t.T


# speedup vs baseline: 3.1305x; 2.2294x over previous
import jax
import jax.numpy as jnp
from jax import lax
from jax.experimental import pallas as pl
from jax.experimental.pallas import tpu as pltpu

_SEQ = 2048
_NOUT = 1000
_NIN = 1000
_TBLK = 1024
_SUBL = _TBLK // 128


def _body(seq_ref, delta_ref, wb_ref, o_ref, srow, wbc, dscr, carry):
    i = pl.program_id(0)

    @pl.when(i == 0)
    def _():
        carry[0, 0] = jnp.float32(0.0)
        wbc[...] = jnp.transpose(wb_ref[...], (1, 0))
        dscr[0:1, 0:_NIN] = delta_ref[...]

    for k in range(_SUBL):
        srow[0:1, k * 128 : (k + 1) * 128] = seq_ref[0, k : k + 1, :]

    dnums = lax.GatherDimensionNumbers(
        offset_dims=(),
        collapsed_slice_dims=(1,),
        start_index_map=(1,),
        operand_batching_dims=(0,),
        start_indices_batching_dims=(0,),
    )
    idx = srow[...]
    idxm = (idx & 127)[:, :, None]
    idxh = idx >> 7
    g = jnp.zeros((1, _TBLK), jnp.float32)
    for c in range(1024 // 128):
        gc = lax.gather(
            dscr[:, c * 128 : (c + 1) * 128],
            idxm,
            dimension_numbers=dnums,
            slice_sizes=(1, 1),
            mode=lax.GatherScatterMode.PROMISE_IN_BOUNDS,
        )
        g = jnp.where(idxh == c, gc, g)

    lane = lax.broadcasted_iota(jnp.int32, (1, _TBLK), 1)
    csum = g
    s = 1
    while s < _TBLK:
        rolled = pltpu.roll(csum, s, 1)
        csum = csum + jnp.where(lane >= s, rolled, jnp.float32(0.0))
        s *= 2
    c0 = carry[0, 0]
    counters = csum + c0
    carry[0, 0] = c0 + jnp.sum(g)

    logits = wbc[:, 0:1] * counters + wbc[:, 1:2]
    m = jnp.max(logits, axis=0, keepdims=True)
    e = jnp.exp(logits - m)
    r = 1.0 / jnp.sum(e, axis=0, keepdims=True)
    o_ref[...] = e * r


def kernel(input_seq, delta, W, b):
    seq3d = input_seq.reshape(_SEQ // _TBLK, _SUBL, 128)
    wbt = jnp.stack([W[:, 0], b])
    out_t = pl.pallas_call(
        _body,
        grid=(_SEQ // _TBLK,),
        in_specs=[
            pl.BlockSpec((1, _SUBL, 128), lambda i: (i, 0, 0)),
            pl.BlockSpec((1, _NIN), lambda i: (0, 0)),
            pl.BlockSpec((2, _NOUT), lambda i: (0, 0)),
        ],
        out_specs=pl.BlockSpec((_NOUT, _TBLK), lambda i: (0, i)),
        out_shape=jax.ShapeDtypeStruct((_NOUT, _SEQ), jnp.float32),
        scratch_shapes=[
            pltpu.VMEM((1, _TBLK), jnp.int32),
            pltpu.VMEM((_NOUT, 2), jnp.float32),
            pltpu.VMEM((1, 1024), jnp.float32),
            pltpu.SMEM((1, 1), jnp.float32),
        ],
    )(seq3d, delta[None, :], wbt)
    return out_t.T


# speedup vs baseline: 3.1661x; 1.0114x over previous
import jax
import jax.numpy as jnp
from jax import lax
from jax.experimental import pallas as pl
from jax.experimental.pallas import tpu as pltpu

_SEQ = 2048
_NOUT = 1000
_NIN = 1000
_TBLK = 512
_SUBL = _TBLK // 128


def _body(seq_ref, delta_ref, w_ref, b_ref, o_ref, srow, wbc, dscr, carry):
    i = pl.program_id(0)

    @pl.when(i == 0)
    def _():
        carry[0, 0] = jnp.float32(0.0)
        wbc[:, 0:1] = jnp.transpose(w_ref[...], (1, 0))
        wbc[:, 1:2] = jnp.transpose(b_ref[...], (1, 0))
        dscr[0:1, 0:_NIN] = delta_ref[...]

    for k in range(_SUBL):
        srow[0:1, k * 128 : (k + 1) * 128] = seq_ref[0, k : k + 1, :]

    dnums = lax.GatherDimensionNumbers(
        offset_dims=(),
        collapsed_slice_dims=(1,),
        start_index_map=(1,),
        operand_batching_dims=(0,),
        start_indices_batching_dims=(0,),
    )
    idx = srow[...]
    idxm = (idx & 127)[:, :, None]
    idxh = idx >> 7
    g = jnp.zeros((1, _TBLK), jnp.float32)
    for c in range(1024 // 128):
        gc = lax.gather(
            dscr[:, c * 128 : (c + 1) * 128],
            idxm,
            dimension_numbers=dnums,
            slice_sizes=(1, 1),
            mode=lax.GatherScatterMode.PROMISE_IN_BOUNDS,
        )
        g = jnp.where(idxh == c, gc, g)

    lane = lax.broadcasted_iota(jnp.int32, (1, _TBLK), 1)
    csum = g
    s = 1
    while s < _TBLK:
        rolled = pltpu.roll(csum, s, 1)
        csum = csum + jnp.where(lane >= s, rolled, jnp.float32(0.0))
        s *= 2
    c0 = carry[0, 0]
    counters = csum + c0
    carry[0, 0] = c0 + jnp.sum(g)

    logits = wbc[:, 0:1] * counters + wbc[:, 1:2]
    m = jnp.max(logits, axis=0, keepdims=True)
    e = jnp.exp(logits - m)
    r = 1.0 / jnp.sum(e, axis=0, keepdims=True)
    o_ref[...] = e * r


def kernel(input_seq, delta, W, b):
    seq3d = input_seq.reshape(_SEQ // _TBLK, _SUBL, 128)
    out_t = pl.pallas_call(
        _body,
        grid=(_SEQ // _TBLK,),
        in_specs=[
            pl.BlockSpec((1, _SUBL, 128), lambda i: (i, 0, 0)),
            pl.BlockSpec((1, _NIN), lambda i: (0, 0)),
            pl.BlockSpec((1, _NOUT), lambda i: (0, 0)),
            pl.BlockSpec((1, _NOUT), lambda i: (0, 0)),
        ],
        out_specs=pl.BlockSpec((_NOUT, _TBLK), lambda i: (0, i)),
        out_shape=jax.ShapeDtypeStruct((_NOUT, _SEQ), jnp.float32),
        scratch_shapes=[
            pltpu.VMEM((1, _TBLK), jnp.int32),
            pltpu.VMEM((_NOUT, 2), jnp.float32),
            pltpu.VMEM((1, 1024), jnp.float32),
            pltpu.SMEM((1, 1), jnp.float32),
        ],
    )(seq3d, delta[None, :], W[:, 0][None, :], b[None, :])
    return out_t.T


# speedup vs baseline: 3.5524x; 1.1220x over previous
import jax
import jax.numpy as jnp
from jax import lax
from jax.experimental import pallas as pl
from jax.experimental.pallas import tpu as pltpu

_SEQ = 2048
_NOUT = 1000
_NIN = 1000
_TBLK = 1024
_SUBL = _TBLK // 128


def _body(seq_ref, delta_ref, w_ref, b_ref, o_ref, srow, wbc, dscr, carry):
    i = pl.program_id(0)

    @pl.when(i == 0)
    def _():
        carry[0, 0] = jnp.float32(0.0)
        wbc[:, 0:1] = jnp.transpose(w_ref[...], (1, 0))
        wbc[:, 1:2] = jnp.transpose(b_ref[...], (1, 0))
        dscr[0:1, 0:_NIN] = delta_ref[...]

    for k in range(_SUBL):
        srow[0:1, k * 128 : (k + 1) * 128] = seq_ref[0, k : k + 1, :]

    dnums = lax.GatherDimensionNumbers(
        offset_dims=(),
        collapsed_slice_dims=(1,),
        start_index_map=(1,),
        operand_batching_dims=(0,),
        start_indices_batching_dims=(0,),
    )
    idx = srow[...]
    idxm = (idx & 127)[:, :, None]
    idxh = idx >> 7
    g = jnp.zeros((1, _TBLK), jnp.float32)
    for c in range(1024 // 128):
        gc = lax.gather(
            dscr[:, c * 128 : (c + 1) * 128],
            idxm,
            dimension_numbers=dnums,
            slice_sizes=(1, 1),
            mode=lax.GatherScatterMode.PROMISE_IN_BOUNDS,
        )
        g = jnp.where(idxh == c, gc, g)

    lane = lax.broadcasted_iota(jnp.int32, (1, _TBLK), 1)
    csum = g
    s = 1
    while s < _TBLK:
        rolled = pltpu.roll(csum, s, 1)
        csum = csum + jnp.where(lane >= s, rolled, jnp.float32(0.0))
        s *= 2
    c0 = carry[0, 0]
    counters = csum + c0
    carry[0, 0] = c0 + jnp.sum(g)

    logits = wbc[:, 0:1] * counters + wbc[:, 1:2]
    m = jnp.max(logits, axis=0, keepdims=True)
    e = jnp.exp(logits - m)
    r = 1.0 / jnp.sum(e, axis=0, keepdims=True)
    o_ref[...] = e * r


def kernel(input_seq, delta, W, b):
    seq3d = input_seq.reshape(_SEQ // _TBLK, _SUBL, 128)
    out_t = pl.pallas_call(
        _body,
        grid=(_SEQ // _TBLK,),
        in_specs=[
            pl.BlockSpec((1, _SUBL, 128), lambda i: (i, 0, 0)),
            pl.BlockSpec((1, _NIN), lambda i: (0, 0)),
            pl.BlockSpec((1, _NOUT), lambda i: (0, 0)),
            pl.BlockSpec((1, _NOUT), lambda i: (0, 0)),
        ],
        out_specs=pl.BlockSpec((_NOUT, _TBLK), lambda i: (0, i)),
        out_shape=jax.ShapeDtypeStruct((_NOUT, _SEQ), jnp.float32),
        scratch_shapes=[
            pltpu.VMEM((1, _TBLK), jnp.int32),
            pltpu.VMEM((_NOUT, 2), jnp.float32),
            pltpu.VMEM((1, 1024), jnp.float32),
            pltpu.SMEM((1, 1), jnp.float32),
        ],
    )(seq3d, delta[None, :], W[:, 0][None, :], b[None, :])
    return out_t.T


# speedup vs baseline: 3.5795x; 1.0076x over previous
import jax
import jax.numpy as jnp
from jax import lax
from jax.experimental import pallas as pl
from jax.experimental.pallas import tpu as pltpu

_SEQ = 2048
_NOUT = 1000
_NIN = 1000
_TBLK = 1024
_SUBL = _TBLK // 128


def _body(seq_ref, delta_ref, w_ref, b_ref, o_ref, srow, wbc, dscr, carry, wstat):
    i = pl.program_id(0)

    @pl.when(i == 0)
    def _():
        carry[0, 0] = jnp.float32(0.0)
        wbc[:, 0:1] = jnp.transpose(w_ref[...], (1, 0))
        wbc[:, 1:2] = jnp.transpose(b_ref[...], (1, 0))
        dscr[0:1, 0:_NIN] = delta_ref[...]
        wstat[0, 0] = jnp.max(w_ref[...])
        wstat[0, 1] = jnp.min(w_ref[...])
        wstat[0, 2] = jnp.max(b_ref[...])

    for k in range(_SUBL):
        srow[0:1, k * 128 : (k + 1) * 128] = seq_ref[0, k : k + 1, :]

    dnums = lax.GatherDimensionNumbers(
        offset_dims=(),
        collapsed_slice_dims=(1,),
        start_index_map=(1,),
        operand_batching_dims=(0,),
        start_indices_batching_dims=(0,),
    )
    idx = srow[...]
    idxm = (idx & 127)[:, :, None]
    idxh = idx >> 7
    g = jnp.zeros((1, _TBLK), jnp.float32)
    for c in range(1024 // 128):
        gc = lax.gather(
            dscr[:, c * 128 : (c + 1) * 128],
            idxm,
            dimension_numbers=dnums,
            slice_sizes=(1, 1),
            mode=lax.GatherScatterMode.PROMISE_IN_BOUNDS,
        )
        g = jnp.where(idxh == c, gc, g)

    lane = lax.broadcasted_iota(jnp.int32, (1, _TBLK), 1)
    csum = g
    s = 1
    while s < _TBLK:
        rolled = pltpu.roll(csum, s, 1)
        csum = csum + jnp.where(lane >= s, rolled, jnp.float32(0.0))
        s *= 2
    c0 = carry[0, 0]
    counters = csum + c0
    carry[0, 0] = c0 + jnp.sum(g)

    mb = jnp.maximum(counters * wstat[0, 0], counters * wstat[0, 1]) + wstat[0, 2]
    e = jnp.exp(wbc[:, 0:1] * counters + (wbc[:, 1:2] - mb))
    s = lax.dot_general(
        jnp.ones((1, _NOUT), jnp.float32), e, (((1,), (0,)), ((), ())),
        preferred_element_type=jnp.float32,
    )
    o_ref[...] = e * (1.0 / s)


def kernel(input_seq, delta, W, b):
    seq3d = input_seq.reshape(_SEQ // _TBLK, _SUBL, 128)
    out_t = pl.pallas_call(
        _body,
        grid=(_SEQ // _TBLK,),
        in_specs=[
            pl.BlockSpec((1, _SUBL, 128), lambda i: (i, 0, 0)),
            pl.BlockSpec((1, _NIN), lambda i: (0, 0)),
            pl.BlockSpec((1, _NOUT), lambda i: (0, 0)),
            pl.BlockSpec((1, _NOUT), lambda i: (0, 0)),
        ],
        out_specs=pl.BlockSpec((_NOUT, _TBLK), lambda i: (0, i)),
        out_shape=jax.ShapeDtypeStruct((_NOUT, _SEQ), jnp.float32),
        scratch_shapes=[
            pltpu.VMEM((1, _TBLK), jnp.int32),
            pltpu.VMEM((_NOUT, 2), jnp.float32),
            pltpu.VMEM((1, 1024), jnp.float32),
            pltpu.SMEM((1, 1), jnp.float32),
            pltpu.SMEM((1, 3), jnp.float32),
        ],
    )(seq3d, delta[None, :], W[:, 0][None, :], b[None, :])
    return out_t.T


# speedup vs baseline: 3.6494x; 1.0195x over previous
import jax
import jax.numpy as jnp
from jax import lax
from jax.experimental import pallas as pl
from jax.experimental.pallas import tpu as pltpu

_SEQ = 2048
_NOUT = 1000
_NIN = 1000
_TBLK = 1024
_SUBL = _TBLK // 128


def _body(seq_ref, delta_ref, w_ref, b_ref, o_ref, srow, wbc, dscr, carry, wstat):
    i = pl.program_id(0)

    @pl.when(i == 0)
    def _():
        carry[0, 0] = jnp.float32(0.0)
        r0 = lax.broadcasted_iota(jnp.int32, (_NOUT, _NOUT), 0)
        c0 = lax.broadcasted_iota(jnp.int32, (_NOUT, _NOUT), 1)
        iden = (r0 == c0).astype(jnp.float32)
        tdims = (((1,), (1,)), ((), ()))
        wbc[:, 0:1] = lax.dot_general(
            iden, w_ref[...], tdims,
            preferred_element_type=jnp.float32,
            precision=lax.Precision.HIGHEST,
        )
        wbc[:, 1:2] = lax.dot_general(
            iden, b_ref[...], tdims,
            preferred_element_type=jnp.float32,
            precision=lax.Precision.HIGHEST,
        )
        dscr[0:1, 0:_NIN] = delta_ref[...]
        wstat[0, 0] = jnp.max(w_ref[...])
        wstat[0, 1] = jnp.min(w_ref[...])
        wstat[0, 2] = jnp.max(b_ref[...])

    for k in range(_SUBL):
        srow[0:1, k * 128 : (k + 1) * 128] = seq_ref[0, k : k + 1, :]

    dnums = lax.GatherDimensionNumbers(
        offset_dims=(),
        collapsed_slice_dims=(1,),
        start_index_map=(1,),
        operand_batching_dims=(0,),
        start_indices_batching_dims=(0,),
    )
    idx = srow[...]
    idxm = (idx & 127)[:, :, None]
    idxh = idx >> 7
    g = jnp.zeros((1, _TBLK), jnp.float32)
    for c in range(1024 // 128):
        gc = lax.gather(
            dscr[:, c * 128 : (c + 1) * 128],
            idxm,
            dimension_numbers=dnums,
            slice_sizes=(1, 1),
            mode=lax.GatherScatterMode.PROMISE_IN_BOUNDS,
        )
        g = jnp.where(idxh == c, gc, g)

    lane = lax.broadcasted_iota(jnp.int32, (1, _TBLK), 1)
    csum = g
    s = 1
    while s < _TBLK:
        rolled = pltpu.roll(csum, s, 1)
        csum = csum + jnp.where(lane >= s, rolled, jnp.float32(0.0))
        s *= 2
    c0 = carry[0, 0]
    counters = csum + c0
    carry[0, 0] = c0 + jnp.sum(g)

    mb = jnp.maximum(counters * wstat[0, 0], counters * wstat[0, 1]) + wstat[0, 2]
    e = jnp.exp(wbc[:, 0:1] * counters + (wbc[:, 1:2] - mb))
    s = lax.dot_general(
        jnp.ones((1, _NOUT), jnp.float32), e, (((1,), (0,)), ((), ())),
        preferred_element_type=jnp.float32,
    )
    o_ref[...] = e * (1.0 / s)


def kernel(input_seq, delta, W, b):
    seq3d = input_seq.reshape(_SEQ // _TBLK, _SUBL, 128)
    out_t = pl.pallas_call(
        _body,
        grid=(_SEQ // _TBLK,),
        in_specs=[
            pl.BlockSpec((1, _SUBL, 128), lambda i: (i, 0, 0)),
            pl.BlockSpec((1, _NIN), lambda i: (0, 0)),
            pl.BlockSpec((1, _NOUT), lambda i: (0, 0)),
            pl.BlockSpec((1, _NOUT), lambda i: (0, 0)),
        ],
        out_specs=pl.BlockSpec((_NOUT, _TBLK), lambda i: (0, i)),
        out_shape=jax.ShapeDtypeStruct((_NOUT, _SEQ), jnp.float32),
        scratch_shapes=[
            pltpu.VMEM((1, _TBLK), jnp.int32),
            pltpu.VMEM((_NOUT, 2), jnp.float32),
            pltpu.VMEM((1, 1024), jnp.float32),
            pltpu.SMEM((1, 1), jnp.float32),
            pltpu.SMEM((1, 3), jnp.float32),
        ],
    )(seq3d, delta[None, :], W[:, 0][None, :], b[None, :])
    return out_t.T
